# Initial kernel scaffold; baseline (speedup 1.0000x reference)
import jax
import jax.numpy as jnp
from jax.experimental import pallas as pl


def kernel(x, one_hot):
    raise NotImplementedError("write your pallas kernel here")


# speedup vs baseline: 1.6951x; 1.6951x over previous
import functools

import jax
import jax.numpy as jnp
from jax import lax
from jax.experimental import pallas as pl
from jax.experimental.pallas import tpu as pltpu
from jax.experimental.pallas import tpu_sc as plsc

B = 1024
F = 128
A = 100
OUT_W = F + A
NUM_TILES = A
SHIFT = B % A
OH_REP = 12

NC = 2
NS = 16
NW = NC * NS


def _sc_body(x_hbm, oh_hbm, out_hbm, x_sp, oh_sp):
    c = lax.axis_index("c")
    s = lax.axis_index("s")
    wid = c * NS + s

    rows_per_s = B // NS
    pltpu.sync_copy(x_hbm.at[pl.ds(s * rows_per_s, rows_per_s)],
                    x_sp.at[pl.ds(s * rows_per_s, rows_per_s)])
    @pl.when(s < OH_REP)
    def _():
        pltpu.sync_copy(oh_hbm, oh_sp.at[pl.ds(s * A, A)])
    plsc.subcore_barrier()

    def do_tile(a):
        row0 = a * B
        start = lax.rem(SHIFT * a, A)
        pltpu.sync_copy(x_sp, out_hbm.at[pl.ds(row0, B), pl.ds(0, F)])
        pltpu.sync_copy(oh_sp.at[pl.ds(start, B)],
                        out_hbm.at[pl.ds(row0, B), pl.ds(F, A)])

    for k in range(NUM_TILES // NW):
        do_tile(wid + NW * k)
    @pl.when(wid < NUM_TILES % NW)
    def _():
        do_tile(wid + NW * (NUM_TILES // NW))


@jax.jit
def _concat_sc(x, one_hot):
    mesh = plsc.VectorSubcoreMesh(core_axis_name="c", subcore_axis_name="s")
    return pl.kernel(
        _sc_body,
        out_type=jax.ShapeDtypeStruct((B * NUM_TILES, OUT_W), jnp.float32),
        mesh=mesh,
        scratch_types=[
            pltpu.VMEM_SHARED((B, F), jnp.float32),
            pltpu.VMEM_SHARED((OH_REP * A, A), jnp.float32),
        ],
    )(x, one_hot)


def kernel(x, one_hot):
    concat_batch = _concat_sc(x, one_hot.astype(x.dtype))
    act = jnp.full((B, A), jnp.nan, dtype=jnp.float32)
    val = jnp.full((B, A), jnp.nan, dtype=jnp.float32)
    mask = jnp.ones((B, A), dtype=bool)
    return (concat_batch, act, val, mask)


# speedup vs baseline: 1.6951x; 1.0000x over previous
import functools

import jax
import jax.numpy as jnp
from jax import lax
from jax.experimental import pallas as pl
from jax.experimental.pallas import tpu as pltpu
from jax.experimental.pallas import tpu_sc as plsc

B = 1024
F = 128
A = 100
OUT_W = F + A
NUM_TILES = A
SHIFT = B % A
OH_REP = 12

NC = 2
NS = 16
NW = NC * NS


def _sc_body(x_hbm, oh_hbm, out_hbm, x_sp, oh_sp, sem):
    c = lax.axis_index("c")
    s = lax.axis_index("s")
    wid = c * NS + s

    rows_per_s = B // NS
    pltpu.sync_copy(x_hbm.at[pl.ds(s * rows_per_s, rows_per_s)],
                    x_sp.at[pl.ds(s * rows_per_s, rows_per_s)])
    @pl.when(s < OH_REP)
    def _():
        pltpu.sync_copy(oh_hbm, oh_sp.at[pl.ds(s * A, A)])
    plsc.subcore_barrier()

    def fire_tile(a):
        row0 = a * B
        start = lax.rem(SHIFT * a, A)
        c1 = pltpu.async_copy(x_sp, out_hbm.at[pl.ds(row0, B), pl.ds(0, F)],
                              sem)
        c2 = pltpu.async_copy(oh_sp.at[pl.ds(start, B)],
                              out_hbm.at[pl.ds(row0, B), pl.ds(F, A)], sem)
        return (c1, c2)

    copies = []
    for k in range(NUM_TILES // NW):
        copies.extend(fire_tile(wid + NW * k))
    for cp in copies:
        cp.wait()
    @pl.when(wid < NUM_TILES % NW)
    def _():
        for cp in fire_tile(wid + NW * (NUM_TILES // NW)):
            cp.wait()


@jax.jit
def _concat_sc(x, one_hot):
    mesh = plsc.VectorSubcoreMesh(core_axis_name="c", subcore_axis_name="s")
    return pl.kernel(
        _sc_body,
        out_type=jax.ShapeDtypeStruct((B * NUM_TILES, OUT_W), jnp.float32),
        mesh=mesh,
        scratch_types=[
            pltpu.VMEM_SHARED((B, F), jnp.float32),
            pltpu.VMEM_SHARED((OH_REP * A, A), jnp.float32),
            pltpu.SemaphoreType.DMA,
        ],
    )(x, one_hot)


def kernel(x, one_hot):
    concat_batch = _concat_sc(x, one_hot.astype(x.dtype))
    act = jnp.full((B, A), jnp.nan, dtype=jnp.float32)
    val = jnp.full((B, A), jnp.nan, dtype=jnp.float32)
    mask = jnp.ones((B, A), dtype=bool)
    return (concat_batch, act, val, mask)


# speedup vs baseline: 1.7822x; 1.0514x over previous
import jax
import jax.numpy as jnp
from jax import lax
from jax.experimental import pallas as pl
from jax.experimental.pallas import tpu as pltpu
from jax.experimental.pallas import tpu_sc as plsc

B = 1024
F = 128
A = 100
OUT_W = F + A
NUM_TILES = A
OH_REP = 2

NC = 2
NS = 16
NW = NC * NS

CHUNK = 64
N_CHUNKS = B * NUM_TILES // CHUNK
CPW = N_CHUNKS // NW


def _sc_body(x_hbm, oh_hbm, out_hbm, x_sp, oh_sp, buf0, buf1, gsem, wsem):
    c = lax.axis_index("c")
    s = lax.axis_index("s")
    wid = c * NS + s

    rows_per_s = B // NS
    pltpu.sync_copy(x_hbm.at[pl.ds(s * rows_per_s, rows_per_s)],
                    x_sp.at[pl.ds(s * rows_per_s, rows_per_s)])
    @pl.when(s < OH_REP)
    def _():
        pltpu.sync_copy(oh_hbm, oh_sp.at[pl.ds(s * A, A)])
    plsc.subcore_barrier()

    bufs = (buf0, buf1)
    writes = [None, None]
    for j in range(CPW):
        p = j & 1
        if writes[p] is not None:
            writes[p].wait()
        g = wid * CPW + j
        xs = lax.rem(CHUNK * g, B)
        os_ = lax.rem(CHUNK * g, A)
        c1 = pltpu.async_copy(x_sp.at[pl.ds(xs, CHUNK)],
                              bufs[p].at[:, pl.ds(0, F)], gsem)
        c2 = pltpu.async_copy(oh_sp.at[pl.ds(os_, CHUNK)],
                              bufs[p].at[:, pl.ds(F, A)], gsem)
        c1.wait()
        c2.wait()
        writes[p] = pltpu.async_copy(
            bufs[p], out_hbm.at[pl.ds(CHUNK * g, CHUNK)], wsem)
    writes[0].wait()
    writes[1].wait()


@jax.jit
def _concat_sc(x, one_hot):
    mesh = plsc.VectorSubcoreMesh(core_axis_name="c", subcore_axis_name="s")
    return pl.kernel(
        _sc_body,
        out_type=jax.ShapeDtypeStruct((B * NUM_TILES, OUT_W), jnp.float32),
        mesh=mesh,
        scratch_types=[
            pltpu.VMEM_SHARED((B, F), jnp.float32),
            pltpu.VMEM_SHARED((OH_REP * A, A), jnp.float32),
            pltpu.VMEM((CHUNK, OUT_W), jnp.float32),
            pltpu.VMEM((CHUNK, OUT_W), jnp.float32),
            pltpu.SemaphoreType.DMA,
            pltpu.SemaphoreType.DMA,
        ],
    )(x, one_hot)


def kernel(x, one_hot):
    concat_batch = _concat_sc(x, one_hot.astype(x.dtype))
    act = jnp.full((B, A), jnp.nan, dtype=jnp.float32)
    val = jnp.full((B, A), jnp.nan, dtype=jnp.float32)
    mask = jnp.ones((B, A), dtype=bool)
    return (concat_batch, act, val, mask)
